# baseline (speedup 1.0000x reference)
import functools

import jax
import jax.numpy as jnp
from jax import lax
from jax.experimental import pallas as pl
from jax.experimental.pallas import tpu as pltpu
from jax.experimental.pallas import tpu_sc as plsc

B, P, C, NT = 32, 8732, 21, 16
RR = 72
PADP = RR * 128
TASKS = B * 3
THRESH = 0.5
NEGPOS = 3
V0, V1 = 0.1, 0.2
THETA = 0.01


def _match_kernel(tg_ref, al_ref, ac_ref, pr_ref, ab_ref):
    i = pl.program_id(0)
    tgb = i * (NT * 5)
    f32 = jnp.float32
    pcx, pcy, pw, ph = pr_ref[0], pr_ref[1], pr_ref[2], pr_ref[3]
    al0, al1, al2, al3 = al_ref[0, 0], al_ref[0, 1], al_ref[0, 2], al_ref[0, 3]
    cx = pcx + al0 * V0 * pw
    cy = pcy + al1 * V0 * ph
    w = pw * jnp.exp(al2 * V1)
    h = ph * jnp.exp(al3 * V1)
    x1 = cx - w * 0.5
    y1 = cy - h * 0.5
    x2 = cx + w * 0.5
    y2 = cy + h * 0.5
    cfx = (x1 + x2) * 0.5
    cfy = (y1 + y2) * 0.5
    cfw = jnp.maximum(x2 - x1, 1e-6)
    cfh = jnp.maximum(y2 - y1, 1e-6)
    area_r = (x2 - x1) * (y2 - y1)

    rows = lax.broadcasted_iota(jnp.int32, (RR, 128), 0)
    cols = lax.broadcasted_iota(jnp.int32, (RR, 128), 1)
    lin = (rows * 128 + cols).astype(f32)
    real = lin < float(P)

    bt_ov = jnp.full((RR, 128), -1.0, f32)
    bt_idx = jnp.zeros((RR, 128), f32)
    truths = []
    bp_lin = []
    for t in range(NT):
        tx1 = tg_ref[tgb + t * 5 + 0]
        ty1 = tg_ref[tgb + t * 5 + 1]
        tx2 = tg_ref[tgb + t * 5 + 2]
        ty2 = tg_ref[tgb + t * 5 + 3]
        tlab = tg_ref[tgb + t * 5 + 4]
        truths.append((tx1, ty1, tx2, ty2, tlab))
        area_t = (tx2 - tx1) * (ty2 - ty1)
        iw = jnp.maximum(jnp.minimum(tx2, x2) - jnp.maximum(tx1, x1), 0.0)
        ih = jnp.maximum(jnp.minimum(ty2, y2) - jnp.maximum(ty1, y1), 0.0)
        inter = iw * ih
        ov = inter / (area_t + area_r - inter + 1e-8)
        ov = jnp.where(real, ov, 0.0)
        upd = ov > bt_ov
        bt_ov = jnp.where(upd, ov, bt_ov)
        bt_idx = jnp.where(upd, float(t), bt_idx)
        m_t = jnp.max(ov)
        bp = jnp.min(jnp.where(ov == m_t, lin, 1e9))
        bp_lin.append(bp)
    for t in range(NT):
        mt = lin == bp_lin[t]
        bt_ov = jnp.where(mt, 2.0, bt_ov)
        bt_idx = jnp.where(mt, float(t), bt_idx)
    conf0 = jnp.zeros((RR, 128), f32)
    mx1 = jnp.zeros((RR, 128), f32)
    my1 = jnp.zeros((RR, 128), f32)
    mx2 = jnp.zeros((RR, 128), f32)
    my2 = jnp.zeros((RR, 128), f32)
    for t in range(NT):
        sel = bt_idx == float(t)
        tx1, ty1, tx2, ty2, tlab = truths[t]
        conf0 = jnp.where(sel, tlab, conf0)
        mx1 = jnp.where(sel, tx1, mx1)
        my1 = jnp.where(sel, ty1, my1)
        mx2 = jnp.where(sel, tx2, mx2)
        my2 = jnp.where(sel, ty2, my2)
    conf_t = jnp.where(bt_ov < THRESH, 0.0, conf0)
    ab_ref[0, 0] = ((mx1 + mx2) * 0.5 - cfx) / (V0 * cfw)
    ab_ref[0, 1] = ((my1 + my2) * 0.5 - cfy) / (V0 * cfh)
    ab_ref[0, 2] = jnp.log(jnp.maximum((mx2 - mx1) / cfw, 1e-8)) / V1
    ab_ref[0, 3] = jnp.log(jnp.maximum((my2 - my1) / cfh, 1e-8)) / V1
    ab_ref[0, 4] = conf_t
    scale = jnp.sqrt(jnp.maximum((mx2 - mx1) * (my2 - my1), 0.0))
    a0, a1 = ac_ref[0, 0], ac_ref[0, 1]
    mx = jnp.maximum(a0, a1)
    e0 = jnp.exp(a0 - mx)
    e1 = jnp.exp(a1 - mx)
    p1 = e1 / (e0 + e1)
    posb = jnp.logical_and(conf_t > 0.0, p1 > THETA)
    masks = (
        jnp.logical_and(posb, scale < 0.1),
        jnp.logical_and(posb, jnp.logical_and(scale >= 0.1, scale < 0.3)),
        jnp.logical_and(posb, scale >= 0.3),
    )
    for sidx in range(3):
        ab_ref[0, 5 + sidx] = jnp.where(masks[sidx], 1.0, 0.0)


def _loss_kernel(ab_ref, tl1_ref, tc1_ref, tl2_ref, tc2_ref,
                 tl3_ref, tc3_ref, mine_ref, kout_ref, acc_out_ref,
                 acc_ref):
    i = pl.program_id(0)

    @pl.when(i == 0)
    def _init():
        for j in range(9):
            acc_ref[j] = 0.0

    f32 = jnp.float32
    rows = lax.broadcasted_iota(jnp.int32, (RR, 128), 0)
    cols = lax.broadcasted_iota(jnp.int32, (RR, 128), 1)
    lin = rows * 128 + cols
    real = lin < P
    gs = (ab_ref[0, 0], ab_ref[0, 1], ab_ref[0, 2], ab_ref[0, 3])
    conf_t = ab_ref[0, 4]
    masks = (ab_ref[0, 5] > 0.5, ab_ref[0, 6] > 0.5, ab_ref[0, 7] > 0.5)
    locs = (tl1_ref, tl2_ref, tl3_ref)
    confs = (tc1_ref, tc2_ref, tc3_ref)

    for s in range(3):
        msk = masks[s]
        ll = jnp.zeros((RR, 128), f32)
        for k in range(4):
            d = locs[s][0, k].astype(f32) - gs[k]
            ad = jnp.abs(d)
            ll = ll + jnp.where(ad < 1.0, 0.5 * d * d, ad - 0.5)
        ll_sum = jnp.sum(jnp.where(msk, ll, 0.0))
        planes = [confs[s][0, c].astype(f32) for c in range(C)]
        m21 = functools.reduce(jnp.maximum, planes)
        se = jnp.zeros((RR, 128), f32)
        for c in range(C):
            se = se + jnp.exp(planes[c] - m21)
        lse = jnp.log(se) + m21
        tgt = jnp.where(msk, conf_t, 0.0)
        picked = jnp.zeros((RR, 128), f32)
        for c in range(C):
            picked = jnp.where(tgt == float(c), planes[c], picked)
        ce = jnp.where(real, lse - picked, 0.0)
        ce_pos = jnp.sum(jnp.where(msk, ce, 0.0))
        npos = jnp.sum(jnp.where(msk, 1.0, 0.0))
        k_f = jnp.minimum(float(NEGPOS) * npos, float(P - 1))
        mine = jnp.where(msk, 0.0, ce)
        mine_ref[0, s] = mine
        kout_ref[0, s] = jnp.full((8, 128), k_f, f32)
        acc_ref[3 * s] = acc_ref[3 * s] + ll_sum
        acc_ref[3 * s + 1] = acc_ref[3 * s + 1] + ce_pos
        acc_ref[3 * s + 2] = acc_ref[3 * s + 2] + npos

    @pl.when(i == B - 1)
    def _fin():
        lane = lax.broadcasted_iota(jnp.int32, (8, 128), 1)
        row = lax.broadcasted_iota(jnp.int32, (8, 128), 0)
        out = jnp.zeros((8, 128), f32)
        for j in range(9):
            out = jnp.where(jnp.logical_and(row == 0, lane == j),
                            acc_ref[j], out)
        acc_out_ref[:, :] = out


_SHIFTS = (23, 15, 7, 0)
_NBUCKETS = (256, 256, 256, 128)


def _make_sc_select():
    mesh = plsc.VectorSubcoreMesh(core_axis_name="c", subcore_axis_name="s")

    @functools.partial(
        pl.kernel,
        out_type=jax.ShapeDtypeStruct((TASKS, 16), jnp.float32),
        mesh=mesh,
        compiler_params=pltpu.CompilerParams(needs_layout_passes=False),
        scratch_types=[
            pltpu.VMEM((PADP,), jnp.float32),
            pltpu.VMEM((4096,), jnp.float32),
            pltpu.VMEM((16,), jnp.float32),
            pltpu.VMEM((16,), jnp.float32),
        ],
    )
    def _sc_select(mine_hbm, k_hbm, out_hbm, mine_v, hist_v, kv_v, res_v):
        cid = lax.axis_index("c")
        sid = lax.axis_index("s")
        wid = sid * 2 + cid
        lanes = lax.broadcasted_iota(jnp.int32, (16,), 0)
        ones = jnp.ones((16,), jnp.float32)
        zeros = jnp.zeros((16,), jnp.float32)

        def bfly_sum(v):
            for d in (1, 2, 4, 8):
                v = v + v[lanes ^ d]
            return v

        def bfly_max(v):
            for d in (1, 2, 4, 8):
                v = jnp.maximum(v, v[lanes ^ d])
            return v

        def suffix_incl(v):
            x = v
            for d in (1, 2, 4, 8):
                y = x[(lanes + d) & 15]
                x = x + jnp.where(lanes < 16 - d, y, jnp.zeros_like(x))
            return x

        for j in range(3):
            task = wid * 3 + j
            pltpu.sync_copy(mine_hbm.at[task], mine_v)
            pltpu.sync_copy(k_hbm.at[pl.ds(task * 1024, 16)], kv_v)
            kf = kv_v[...]
            prefix = jnp.zeros((16,), jnp.int32)
            above = zeros
            for lvl in range(4):
                sh = _SHIFTS[lvl]
                nb = _NBUCKETS[lvl]
                hs = sh + (8 if lvl < 3 else 7)

                def zbody(z, _):
                    for u in range(4):
                        hist_v[pl.ds((z * 4 + u) * 16, 16)] = zeros
                    return 0

                lax.fori_loop(0, nb // 4, zbody, 0)

                pref_hi = lax.shift_right_logical(prefix, hs)

                def pbody(v, _, sh=sh, nb=nb, hs=hs, pref_hi=pref_hi):
                    for u in range(8):
                        g = v * 8 + u
                        vec = mine_v[pl.ds(g * 16, 16)]
                        xi = lax.bitcast_convert_type(vec, jnp.int32)
                        dig = lax.shift_right_logical(xi, sh) & (nb - 1)
                        m = lax.shift_right_logical(xi, hs) == pref_hi
                        idx = lanes * nb + dig
                        plsc.addupdate_scatter(hist_v, [idx], ones, mask=m)
                    return 0

                lax.fori_loop(0, PADP // 128, pbody, 0)

                rem = kf - above
                nchunks = nb // 16

                def sbody(r, carry, nb=nb, nchunks=nchunks, rem=rem):
                    carry_cnt, td_v, ab_v = carry
                    base = (nchunks - 1 - r) * 16
                    cv = hist_v[pl.ds(base, 16)]
                    for bk in range(1, 16):
                        cv = cv + hist_v[pl.ds(bk * nb + base, 16)]
                    incl = suffix_incl(cv)
                    se = incl - cv
                    total = incl[jnp.zeros((16,), jnp.int32)]
                    ca_i = carry_cnt + incl
                    ca_e = carry_cnt + se
                    cross = jnp.logical_and(ca_i >= rem, ca_e < rem)
                    bid = base + lanes
                    tdc = bfly_max(jnp.where(cross, bid, -1))
                    abc = bfly_max(jnp.where(cross, ca_e, -1.0))
                    found = tdc >= 0
                    td_v = jnp.where(found, tdc, td_v)
                    ab_v = jnp.where(found, abc, ab_v)
                    return (carry_cnt + total, td_v, ab_v)

                _, td_v, ab_v = lax.fori_loop(
                    0, nchunks, sbody,
                    (zeros, jnp.full((16,), -1, jnp.int32), zeros))
                above = above + jnp.where(td_v >= 0, ab_v, zeros)
                prefix = prefix | lax.shift_left(jnp.maximum(td_v, 0),
                                                 jnp.full((16,), sh, jnp.int32))

            tbits = prefix

            def fbody(v, carry, tbits=tbits):
                cgt, sgt = carry
                for u in range(8):
                    g = v * 8 + u
                    vec = mine_v[pl.ds(g * 16, 16)]
                    xi = lax.bitcast_convert_type(vec, jnp.int32)
                    gt = xi > tbits
                    cgt = cgt + jnp.where(gt, 1.0, 0.0)
                    sgt = sgt + jnp.where(gt, vec, 0.0)
                return (cgt, sgt)

            cgt, sgt = lax.fori_loop(0, PADP // 128, fbody, (zeros, zeros))
            cnt_gt = bfly_sum(cgt)
            sum_gt = bfly_sum(sgt)
            tval = lax.bitcast_convert_type(tbits, jnp.float32)
            topk = sum_gt + (kf - cnt_gt) * tval
            topk = jnp.where(kf > 0.0, topk, zeros)
            res_v[...] = jnp.where(lanes == 0, topk, zeros)
            pltpu.sync_copy(res_v, out_hbm.at[task])

    return _sc_select


_sc_select = None


def _to_planes(x, n, dtype=None):
    xb = x.shape[0]
    if dtype is not None:
        x = x.astype(dtype)
    x = jnp.moveaxis(x, -1, 1)
    x = jnp.pad(x, ((0, 0), (0, 0), (0, PADP - P)))
    return x.reshape(xb, n, RR, 128)


def kernel(arm_loc, arm_conf, trm_loc1, trm_conf1, trm_loc2, trm_conf2,
           trm_loc3, trm_conf3, priors, targets):
    al = _to_planes(arm_loc, 4)
    ac = _to_planes(arm_conf, 2)
    tl1 = _to_planes(trm_loc1, 4, jnp.bfloat16)
    tl2 = _to_planes(trm_loc2, 4, jnp.bfloat16)
    tl3 = _to_planes(trm_loc3, 4, jnp.bfloat16)
    tc1 = _to_planes(trm_conf1, C, jnp.bfloat16)
    tc2 = _to_planes(trm_conf2, C, jnp.bfloat16)
    tc3 = _to_planes(trm_conf3, C, jnp.bfloat16)
    pr = jnp.pad(priors.T, ((0, 0), (0, PADP - P))).reshape(4, RR, 128)

    def img_spec(n):
        return pl.BlockSpec((1, n, RR, 128), lambda i, tg: (i, 0, 0, 0))

    ab = pl.pallas_call(
        _match_kernel,
        grid_spec=pltpu.PrefetchScalarGridSpec(
            num_scalar_prefetch=1,
            grid=(B,),
            in_specs=[
                img_spec(4), img_spec(2),
                pl.BlockSpec((4, RR, 128), lambda i, tg: (0, 0, 0)),
            ],
            out_specs=pl.BlockSpec((1, 8, RR, 128), lambda i, tg: (i, 0, 0, 0)),
        ),
        out_shape=jax.ShapeDtypeStruct((B, 8, RR, 128), jnp.float32),
    )(targets.reshape(B * NT * 5), al, ac, pr)

    def img_spec2(n):
        return pl.BlockSpec((1, n, RR, 128), lambda i: (i, 0, 0, 0))

    mine4, kacc, accout = pl.pallas_call(
        _loss_kernel,
        grid=(B,),
        in_specs=[
            img_spec2(8),
            img_spec2(4), img_spec2(C),
            img_spec2(4), img_spec2(C),
            img_spec2(4), img_spec2(C),
        ],
        out_specs=[
            pl.BlockSpec((1, 3, RR, 128), lambda i: (i, 0, 0, 0)),
            pl.BlockSpec((1, 3, 8, 128), lambda i: (i, 0, 0, 0)),
            pl.BlockSpec((8, 128), lambda i: (0, 0)),
        ],
        out_shape=[
            jax.ShapeDtypeStruct((B, 3, RR, 128), jnp.float32),
            jax.ShapeDtypeStruct((B, 3, 8, 128), jnp.float32),
            jax.ShapeDtypeStruct((8, 128), jnp.float32),
        ],
        scratch_shapes=[pltpu.SMEM((16,), jnp.float32)],
    )(ab, tl1, tc1, tl2, tc2, tl3, tc3)

    global _sc_select
    if _sc_select is None:
        _sc_select = _make_sc_select()
    minef = mine4.reshape(TASKS, PADP)
    kflat = kacc.reshape(TASKS * 1024)
    topk_out = _sc_select(minef, kflat)

    a = accout[0]
    tk = jnp.sum(topk_out[:, 0].reshape(B, 3), axis=0)
    loss_l = jnp.float32(0.0)
    loss_c = jnp.float32(0.0)
    for s in range(3):
        n = jnp.maximum(a[3 * s + 2], 1.0)
        loss_l = loss_l + a[3 * s] / n
        loss_c = loss_c + (a[3 * s + 1] + tk[s]) / n
    return jnp.stack([loss_l, loss_c])


# speedup vs baseline: 1.1377x; 1.1377x over previous
import functools

import jax
import jax.numpy as jnp
from jax import lax
from jax.experimental import pallas as pl
from jax.experimental.pallas import tpu as pltpu
from jax.experimental.pallas import tpu_sc as plsc

B, P, C, NT = 32, 8732, 21, 16
RR = 72
PADP = RR * 128
TASKS = B * 3
THRESH = 0.5
NEGPOS = 3
V0, V1 = 0.1, 0.2
THETA = 0.01


def _match_kernel(tg_ref, al_ref, ac_ref, pr_ref, ab_ref, ovs_ref):
    i = pl.program_id(0)
    tgb = i * (NT * 5)
    f32 = jnp.float32
    pcx, pcy, pw, ph = pr_ref[0], pr_ref[1], pr_ref[2], pr_ref[3]
    al0, al1, al2, al3 = al_ref[0, 0], al_ref[0, 1], al_ref[0, 2], al_ref[0, 3]
    cx = pcx + al0 * V0 * pw
    cy = pcy + al1 * V0 * ph
    w = pw * jnp.exp(al2 * V1)
    h = ph * jnp.exp(al3 * V1)
    x1 = cx - w * 0.5
    y1 = cy - h * 0.5
    x2 = cx + w * 0.5
    y2 = cy + h * 0.5
    cfx = (x1 + x2) * 0.5
    cfy = (y1 + y2) * 0.5
    cfw = jnp.maximum(x2 - x1, 1e-6)
    cfh = jnp.maximum(y2 - y1, 1e-6)
    area_r = (x2 - x1) * (y2 - y1)

    rows = lax.broadcasted_iota(jnp.int32, (RR, 128), 0)
    cols = lax.broadcasted_iota(jnp.int32, (RR, 128), 1)
    lin = (rows * 128 + cols).astype(f32)
    real = lin < float(P)

    bt_ov = jnp.full((RR, 128), -1.0, f32)
    bt_idx = jnp.zeros((RR, 128), f32)
    truths = []
    pms = []
    for t in range(NT):
        tx1 = tg_ref[tgb + t * 5 + 0]
        ty1 = tg_ref[tgb + t * 5 + 1]
        tx2 = tg_ref[tgb + t * 5 + 2]
        ty2 = tg_ref[tgb + t * 5 + 3]
        tlab = tg_ref[tgb + t * 5 + 4]
        truths.append((tx1, ty1, tx2, ty2, tlab))
        area_t = (tx2 - tx1) * (ty2 - ty1)
        iw = jnp.maximum(jnp.minimum(tx2, x2) - jnp.maximum(tx1, x1), 0.0)
        ih = jnp.maximum(jnp.minimum(ty2, y2) - jnp.maximum(ty1, y1), 0.0)
        inter = iw * ih
        ov = inter / (area_t + area_r - inter + 1e-8)
        ov = jnp.where(real, ov, 0.0)
        ovs_ref[t] = ov
        upd = ov > bt_ov
        bt_ov = jnp.where(upd, ov, bt_ov)
        bt_idx = jnp.where(upd, float(t), bt_idx)
        pms.append(functools.reduce(jnp.maximum,
                                    [ov[8 * r:8 * r + 8] for r in range(9)]))
    m_ts = [jnp.max(pms[t]) for t in range(NT)]
    bp_lin = [jnp.min(jnp.where(ovs_ref[t] == m_ts[t], lin, 1e9))
              for t in range(NT)]
    for t in range(NT):
        mt = lin == bp_lin[t]
        bt_ov = jnp.where(mt, 2.0, bt_ov)
        bt_idx = jnp.where(mt, float(t), bt_idx)
    conf0 = jnp.zeros((RR, 128), f32)
    mx1 = jnp.zeros((RR, 128), f32)
    my1 = jnp.zeros((RR, 128), f32)
    mx2 = jnp.zeros((RR, 128), f32)
    my2 = jnp.zeros((RR, 128), f32)
    for t in range(NT):
        sel = bt_idx == float(t)
        tx1, ty1, tx2, ty2, tlab = truths[t]
        conf0 = jnp.where(sel, tlab, conf0)
        mx1 = jnp.where(sel, tx1, mx1)
        my1 = jnp.where(sel, ty1, my1)
        mx2 = jnp.where(sel, tx2, mx2)
        my2 = jnp.where(sel, ty2, my2)
    conf_t = jnp.where(bt_ov < THRESH, 0.0, conf0)
    ab_ref[0, 0] = ((mx1 + mx2) * 0.5 - cfx) / (V0 * cfw)
    ab_ref[0, 1] = ((my1 + my2) * 0.5 - cfy) / (V0 * cfh)
    ab_ref[0, 2] = jnp.log(jnp.maximum((mx2 - mx1) / cfw, 1e-8)) / V1
    ab_ref[0, 3] = jnp.log(jnp.maximum((my2 - my1) / cfh, 1e-8)) / V1
    ab_ref[0, 4] = conf_t
    scale = jnp.sqrt(jnp.maximum((mx2 - mx1) * (my2 - my1), 0.0))
    a0, a1 = ac_ref[0, 0], ac_ref[0, 1]
    mx = jnp.maximum(a0, a1)
    e0 = jnp.exp(a0 - mx)
    e1 = jnp.exp(a1 - mx)
    p1 = e1 / (e0 + e1)
    posb = jnp.logical_and(conf_t > 0.0, p1 > THETA)
    masks = (
        jnp.logical_and(posb, scale < 0.1),
        jnp.logical_and(posb, jnp.logical_and(scale >= 0.1, scale < 0.3)),
        jnp.logical_and(posb, scale >= 0.3),
    )
    for sidx in range(3):
        ab_ref[0, 5 + sidx] = jnp.where(masks[sidx], 1.0, 0.0)


def _loss_kernel(ab_ref, tl1_ref, tc1_ref, tl2_ref, tc2_ref,
                 tl3_ref, tc3_ref, mine_ref, kout_ref, acc_out_ref,
                 acc_ref):
    i = pl.program_id(0)

    @pl.when(i == 0)
    def _init():
        for j in range(9):
            acc_ref[j] = 0.0

    f32 = jnp.float32
    rows = lax.broadcasted_iota(jnp.int32, (RR, 128), 0)
    cols = lax.broadcasted_iota(jnp.int32, (RR, 128), 1)
    lin = rows * 128 + cols
    real = lin < P
    gs = (ab_ref[0, 0], ab_ref[0, 1], ab_ref[0, 2], ab_ref[0, 3])
    conf_t = ab_ref[0, 4]
    masks = (ab_ref[0, 5] > 0.5, ab_ref[0, 6] > 0.5, ab_ref[0, 7] > 0.5)
    locs = (tl1_ref, tl2_ref, tl3_ref)
    confs = (tc1_ref, tc2_ref, tc3_ref)

    for s in range(3):
        msk = masks[s]
        ll = jnp.zeros((RR, 128), f32)
        for k in range(4):
            d = locs[s][0, k].astype(f32) - gs[k]
            ad = jnp.abs(d)
            ll = ll + jnp.where(ad < 1.0, 0.5 * d * d, ad - 0.5)
        ll_sum = jnp.sum(jnp.where(msk, ll, 0.0))
        planes = [confs[s][0, c].astype(f32) for c in range(C)]
        m21 = functools.reduce(jnp.maximum, planes)
        se = jnp.zeros((RR, 128), f32)
        for c in range(C):
            se = se + jnp.exp(planes[c] - m21)
        lse = jnp.log(se) + m21
        tgt = jnp.where(msk, conf_t, 0.0)
        picked = jnp.zeros((RR, 128), f32)
        for c in range(C):
            picked = jnp.where(tgt == float(c), planes[c], picked)
        ce = jnp.where(real, lse - picked, 0.0)
        ce_pos = jnp.sum(jnp.where(msk, ce, 0.0))
        npos = jnp.sum(jnp.where(msk, 1.0, 0.0))
        k_f = jnp.minimum(float(NEGPOS) * npos, float(P - 1))
        mine = jnp.where(msk, 0.0, ce)
        mine_ref[0, s] = mine
        kout_ref[0, s] = jnp.full((8, 128), k_f, f32)
        acc_ref[3 * s] = acc_ref[3 * s] + ll_sum
        acc_ref[3 * s + 1] = acc_ref[3 * s + 1] + ce_pos
        acc_ref[3 * s + 2] = acc_ref[3 * s + 2] + npos

    @pl.when(i == B - 1)
    def _fin():
        lane = lax.broadcasted_iota(jnp.int32, (8, 128), 1)
        row = lax.broadcasted_iota(jnp.int32, (8, 128), 0)
        out = jnp.zeros((8, 128), f32)
        for j in range(9):
            out = jnp.where(jnp.logical_and(row == 0, lane == j),
                            acc_ref[j], out)
        acc_out_ref[:, :] = out


_SHIFTS = (23, 15, 7, 0)
_NBUCKETS = (256, 256, 256, 128)


def _make_sc_select():
    mesh = plsc.VectorSubcoreMesh(core_axis_name="c", subcore_axis_name="s")

    @functools.partial(
        pl.kernel,
        out_type=jax.ShapeDtypeStruct((TASKS, 16), jnp.float32),
        mesh=mesh,
        compiler_params=pltpu.CompilerParams(needs_layout_passes=False),
        scratch_types=[
            pltpu.VMEM((PADP,), jnp.float32),
            pltpu.VMEM((4096,), jnp.float32),
            pltpu.VMEM((16,), jnp.float32),
            pltpu.VMEM((16,), jnp.float32),
        ],
    )
    def _sc_select(mine_hbm, k_hbm, out_hbm, mine_v, hist_v, kv_v, res_v):
        cid = lax.axis_index("c")
        sid = lax.axis_index("s")
        wid = sid * 2 + cid
        lanes = lax.broadcasted_iota(jnp.int32, (16,), 0)
        ones = jnp.ones((16,), jnp.float32)
        zeros = jnp.zeros((16,), jnp.float32)

        def bfly_sum(v):
            for d in (1, 2, 4, 8):
                v = v + v[lanes ^ d]
            return v

        def bfly_max(v):
            for d in (1, 2, 4, 8):
                v = jnp.maximum(v, v[lanes ^ d])
            return v

        def suffix_incl(v):
            x = v
            for d in (1, 2, 4, 8):
                y = x[(lanes + d) & 15]
                x = x + jnp.where(lanes < 16 - d, y, jnp.zeros_like(x))
            return x

        for j in range(3):
            task = wid * 3 + j
            pltpu.sync_copy(mine_hbm.at[task], mine_v)
            pltpu.sync_copy(k_hbm.at[pl.ds(task * 1024, 16)], kv_v)
            kf = kv_v[...]
            prefix = jnp.zeros((16,), jnp.int32)
            above = zeros
            for lvl in range(4):
                sh = _SHIFTS[lvl]
                nb = _NBUCKETS[lvl]
                hs = sh + (8 if lvl < 3 else 7)

                def zbody(z, _):
                    for u in range(4):
                        hist_v[pl.ds((z * 4 + u) * 16, 16)] = zeros
                    return 0

                lax.fori_loop(0, nb // 4, zbody, 0)

                pref_hi = lax.shift_right_logical(prefix, hs)

                def pbody(v, _, sh=sh, nb=nb, hs=hs, pref_hi=pref_hi):
                    for u in range(8):
                        g = v * 8 + u
                        vec = mine_v[pl.ds(g * 16, 16)]
                        xi = lax.bitcast_convert_type(vec, jnp.int32)
                        dig = lax.shift_right_logical(xi, sh) & (nb - 1)
                        m = lax.shift_right_logical(xi, hs) == pref_hi
                        idx = lanes * nb + dig
                        plsc.addupdate_scatter(hist_v, [idx], ones, mask=m)
                    return 0

                lax.fori_loop(0, PADP // 128, pbody, 0)

                rem = kf - above
                nchunks = nb // 16

                def sbody(r, carry, nb=nb, nchunks=nchunks, rem=rem):
                    carry_cnt, td_v, ab_v = carry
                    base = (nchunks - 1 - r) * 16
                    cv = hist_v[pl.ds(base, 16)]
                    for bk in range(1, 16):
                        cv = cv + hist_v[pl.ds(bk * nb + base, 16)]
                    incl = suffix_incl(cv)
                    se = incl - cv
                    total = incl[jnp.zeros((16,), jnp.int32)]
                    ca_i = carry_cnt + incl
                    ca_e = carry_cnt + se
                    cross = jnp.logical_and(ca_i >= rem, ca_e < rem)
                    bid = base + lanes
                    tdc = bfly_max(jnp.where(cross, bid, -1))
                    abc = bfly_max(jnp.where(cross, ca_e, -1.0))
                    found = tdc >= 0
                    td_v = jnp.where(found, tdc, td_v)
                    ab_v = jnp.where(found, abc, ab_v)
                    return (carry_cnt + total, td_v, ab_v)

                _, td_v, ab_v = lax.fori_loop(
                    0, nchunks, sbody,
                    (zeros, jnp.full((16,), -1, jnp.int32), zeros))
                above = above + jnp.where(td_v >= 0, ab_v, zeros)
                prefix = prefix | lax.shift_left(jnp.maximum(td_v, 0),
                                                 jnp.full((16,), sh, jnp.int32))

            tbits = prefix

            def fbody(v, carry, tbits=tbits):
                cgt, sgt = carry
                for u in range(8):
                    g = v * 8 + u
                    vec = mine_v[pl.ds(g * 16, 16)]
                    xi = lax.bitcast_convert_type(vec, jnp.int32)
                    gt = xi > tbits
                    cgt = cgt + jnp.where(gt, 1.0, 0.0)
                    sgt = sgt + jnp.where(gt, vec, 0.0)
                return (cgt, sgt)

            cgt, sgt = lax.fori_loop(0, PADP // 128, fbody, (zeros, zeros))
            cnt_gt = bfly_sum(cgt)
            sum_gt = bfly_sum(sgt)
            tval = lax.bitcast_convert_type(tbits, jnp.float32)
            topk = sum_gt + (kf - cnt_gt) * tval
            topk = jnp.where(kf > 0.0, topk, zeros)
            res_v[...] = jnp.where(lanes == 0, topk, zeros)
            pltpu.sync_copy(res_v, out_hbm.at[task])

    return _sc_select


_sc_select = None


def _to_planes(x, n, dtype=None):
    xb = x.shape[0]
    if dtype is not None:
        x = x.astype(dtype)
    x = jnp.moveaxis(x, -1, 1)
    x = jnp.pad(x, ((0, 0), (0, 0), (0, PADP - P)))
    return x.reshape(xb, n, RR, 128)


def kernel(arm_loc, arm_conf, trm_loc1, trm_conf1, trm_loc2, trm_conf2,
           trm_loc3, trm_conf3, priors, targets):
    al = _to_planes(arm_loc, 4)
    ac = _to_planes(arm_conf, 2)
    tl1 = _to_planes(trm_loc1, 4, jnp.bfloat16)
    tl2 = _to_planes(trm_loc2, 4, jnp.bfloat16)
    tl3 = _to_planes(trm_loc3, 4, jnp.bfloat16)
    tc1 = _to_planes(trm_conf1, C, jnp.bfloat16)
    tc2 = _to_planes(trm_conf2, C, jnp.bfloat16)
    tc3 = _to_planes(trm_conf3, C, jnp.bfloat16)
    pr = jnp.pad(priors.T, ((0, 0), (0, PADP - P))).reshape(4, RR, 128)

    def img_spec(n):
        return pl.BlockSpec((1, n, RR, 128), lambda i, tg: (i, 0, 0, 0))

    ab = pl.pallas_call(
        _match_kernel,
        grid_spec=pltpu.PrefetchScalarGridSpec(
            num_scalar_prefetch=1,
            grid=(B,),
            in_specs=[
                img_spec(4), img_spec(2),
                pl.BlockSpec((4, RR, 128), lambda i, tg: (0, 0, 0)),
            ],
            out_specs=pl.BlockSpec((1, 8, RR, 128), lambda i, tg: (i, 0, 0, 0)),
            scratch_shapes=[pltpu.VMEM((NT, RR, 128), jnp.float32)],
        ),
        out_shape=jax.ShapeDtypeStruct((B, 8, RR, 128), jnp.float32),
    )(targets.reshape(B * NT * 5), al, ac, pr)

    def img_spec2(n):
        return pl.BlockSpec((1, n, RR, 128), lambda i: (i, 0, 0, 0))

    mine4, kacc, accout = pl.pallas_call(
        _loss_kernel,
        grid=(B,),
        in_specs=[
            img_spec2(8),
            img_spec2(4), img_spec2(C),
            img_spec2(4), img_spec2(C),
            img_spec2(4), img_spec2(C),
        ],
        out_specs=[
            pl.BlockSpec((1, 3, RR, 128), lambda i: (i, 0, 0, 0)),
            pl.BlockSpec((1, 3, 8, 128), lambda i: (i, 0, 0, 0)),
            pl.BlockSpec((8, 128), lambda i: (0, 0)),
        ],
        out_shape=[
            jax.ShapeDtypeStruct((B, 3, RR, 128), jnp.float32),
            jax.ShapeDtypeStruct((B, 3, 8, 128), jnp.float32),
            jax.ShapeDtypeStruct((8, 128), jnp.float32),
        ],
        scratch_shapes=[pltpu.SMEM((16,), jnp.float32)],
    )(ab, tl1, tc1, tl2, tc2, tl3, tc3)

    global _sc_select
    if _sc_select is None:
        _sc_select = _make_sc_select()
    minef = mine4.reshape(TASKS, PADP)
    kflat = kacc.reshape(TASKS * 1024)
    topk_out = _sc_select(minef, kflat)

    a = accout[0]
    tk = jnp.sum(topk_out[:, 0].reshape(B, 3), axis=0)
    loss_l = jnp.float32(0.0)
    loss_c = jnp.float32(0.0)
    for s in range(3):
        n = jnp.maximum(a[3 * s + 2], 1.0)
        loss_l = loss_l + a[3 * s] / n
        loss_c = loss_c + (a[3 * s + 1] + tk[s]) / n
    return jnp.stack([loss_l, loss_c])
